# baseline (speedup 1.0000x reference)
import functools

import jax
import jax.numpy as jnp
from jax import lax
from jax.experimental import pallas as pl
from jax.experimental.pallas import tpu as pltpu
from jax.experimental.pallas import tpu_sc as plsc

B = 16384
D = 32
NC = 2
NS = 16
NW = NC * NS
BPW = B // NW
CHUNK = 128
NCH = BPW // CHUNK


def _sc_gather(uid_idx, iid_idx, uid_table, iid_table):
    mesh = plsc.VectorSubcoreMesh(
        core_axis_name="c", subcore_axis_name="s",
        num_cores=NC, num_subcores=NS)

    @functools.partial(
        pl.kernel, mesh=mesh,
        compiler_params=pltpu.CompilerParams(use_tc_tiling_on_sc=False),
        out_type=(jax.ShapeDtypeStruct((B, D), jnp.float32),
                  jax.ShapeDtypeStruct((B, D), jnp.float32)),
        scratch_types=[
            pltpu.VMEM((NCH, CHUNK), jnp.int32),
            pltpu.VMEM((NCH, CHUNK), jnp.int32),
            pltpu.VMEM((BPW, D), jnp.float32),
            pltpu.VMEM((BPW, D), jnp.float32),
            pltpu.SemaphoreType.DMA,
        ],
    )
    def k(uidx_hbm, iidx_hbm, utab, itab, uout, iout,
          uidx_v, iidx_v, urows, irows, sem):
        wid = lax.axis_index("s") * NC + lax.axis_index("c")
        base = wid * BPW
        pltpu.sync_copy(uidx_hbm.at[wid], uidx_v)
        pltpu.sync_copy(iidx_hbm.at[wid], iidx_v)
        descs = []
        for j in range(NCH):
            descs.append(pltpu.async_copy(
                utab.at[uidx_v.at[j]],
                urows.at[pl.ds(j * CHUNK, CHUNK)], sem))
            descs.append(pltpu.async_copy(
                itab.at[iidx_v.at[j]],
                irows.at[pl.ds(j * CHUNK, CHUNK)], sem))
        for dsc in descs:
            dsc.wait()
        pltpu.sync_copy(urows, uout.at[pl.ds(base, BPW)])
        pltpu.sync_copy(irows, iout.at[pl.ds(base, BPW)])

    return k(uid_idx, iid_idx, uid_table, iid_table)


def _tc_combine(urows, irows, wt, b2):
    def body(u_ref, i_ref, wt_ref, b_ref, o_ref):
        proj = jnp.dot(u_ref[...], wt_ref[...],
                       preferred_element_type=jnp.float32) + b_ref[...]
        o_ref[...] = jnp.sum(proj * i_ref[...], axis=1)

    return pl.pallas_call(
        body,
        out_shape=jax.ShapeDtypeStruct((B,), jnp.float32),
    )(urows, irows, wt, b2)


def kernel(x, uid_table, iid_table, W, b):
    uid_idx = x[:, 0].reshape(NW, NCH, CHUNK)
    iid_idx = x[:, 1].reshape(NW, NCH, CHUNK)
    urows, irows = _sc_gather(uid_idx, iid_idx, uid_table, iid_table)
    return _tc_combine(urows, irows, W.T, b.reshape(1, D))
